# baseline (speedup 1.0000x reference)
import jax
import jax.numpy as jnp
from jax import lax
from jax.experimental import pallas as pl
from jax.experimental.pallas import tpu as pltpu
from jax.experimental.pallas import tpu_sc as plsc

SAMPLE_RATE = 16000
BLOCK = 64
MAX_DELAY = 480
N_VOICES = 16 * 512
NUM_WORKERS = 32
VPW = N_VOICES // NUM_WORKERS
GROUPS = VPW // 16
UNROLL = 4


def _sc_body(noise_hbm, params_hbm, out_hbm, noise_v, out_v, params_v):
    wid = lax.axis_index("s") * 2 + lax.axis_index("c")
    pltpu.sync_copy(noise_hbm.at[wid], noise_v)
    pltpu.sync_copy(params_hbm.at[wid], params_v)

    def group(g, _):
        vi = g * 16 + lax.iota(jnp.int32, 16)
        pidx = vi * 4
        pa = plsc.load_gather(params_v, [pidx])
        en = plsc.load_gather(params_v, [pidx + 1])
        pt = plsc.load_gather(params_v, [pidx + 2])
        pb = plsc.load_gather(params_v, [pidx + 3])
        tau = jnp.maximum((0.0005 + pa * 0.0495) * SAMPLE_RATE, 1.0)
        rho = jnp.exp(-1.0 / tau)
        rho2 = rho * rho
        rho4 = rho2 * rho2
        tilt = pt * 2.0 - 1.0
        bandwidth = 0.05 + pb * 0.95
        dly = jnp.clip((BLOCK * (0.5 + 0.5 * bandwidth)).astype(jnp.int32),
                       1, MAX_DELAY)
        cb = vi * BLOCK

        def tap(gi):
            msk = gi >= cb
            prev = plsc.load_gather(out_v, [jnp.maximum(gi, cb)], mask=msk)
            return jnp.where(msk, prev, 0.0)

        def step4(k, carry):
            env, acc0, acc1, nidx, gidx = carry
            e0 = env
            e1 = env * rho
            e2 = env * rho2
            e3 = e1 * rho2
            n0, n1, n2, n3 = nidx, nidx + 1, nidx + 2, nidx + 3
            g0, g1, g2, g3 = gidx, gidx + 1, gidx + 2, gidx + 3
            y0 = plsc.load_gather(noise_v, [n0]) * e0 + tilt * tap(g0)
            plsc.store_scatter(out_v, [n0], y0)
            y1 = plsc.load_gather(noise_v, [n1]) * e1 + tilt * tap(g1)
            plsc.store_scatter(out_v, [n1], y1)
            y2 = plsc.load_gather(noise_v, [n2]) * e2 + tilt * tap(g2)
            plsc.store_scatter(out_v, [n2], y2)
            y3 = plsc.load_gather(noise_v, [n3]) * e3 + tilt * tap(g3)
            plsc.store_scatter(out_v, [n3], y3)
            acc0 = acc0 + y0 * y0 + y1 * y1
            acc1 = acc1 + y2 * y2 + y3 * y3
            return (env * rho4, acc0, acc1, nidx + 4, gidx + 4)

        zf = jnp.zeros((16,), jnp.float32)
        _, acc0, acc1, _, _ = lax.fori_loop(
            0, BLOCK // UNROLL, step4, (en, zf, zf, cb, cb - dly))

        m = (acc0 + acc1) * (1.0 / BLOCK) + 1e-5
        bits = plsc.bitcast(m, jnp.int32)
        r = plsc.bitcast(0x5F3759DF - (bits >> 1), jnp.float32)
        for _ in range(4):
            r = r * (1.5 - 0.5 * m * r * r)

        def scale4(k, nidx):
            for j in range(UNROLL):
                nj = nidx + j
                v = plsc.load_gather(out_v, [nj])
                plsc.store_scatter(out_v, [nj], v * r)
            return nidx + UNROLL

        lax.fori_loop(0, BLOCK // UNROLL, scale4, cb)
        return 0

    lax.fori_loop(0, GROUPS, group, 0)
    pltpu.sync_copy(out_v, out_hbm.at[wid])


_sc_call = pl.kernel(
    _sc_body,
    out_type=jax.ShapeDtypeStruct((NUM_WORKERS, VPW * BLOCK), jnp.float32),
    mesh=plsc.VectorSubcoreMesh(core_axis_name="c", subcore_axis_name="s"),
    compiler_params=pltpu.CompilerParams(needs_layout_passes=False),
    scratch_types=[
        pltpu.VMEM((VPW * BLOCK,), jnp.float32),
        pltpu.VMEM((VPW * BLOCK,), jnp.float32),
        pltpu.VMEM((VPW * 4,), jnp.float32),
    ],
)


def kernel(transient_params, noise):
    Bb, Tt, _ = transient_params.shape
    p = transient_params.reshape(NUM_WORKERS, VPW * 4)
    noise_rows = noise.reshape(NUM_WORKERS, VPW * BLOCK)
    out = _sc_call(noise_rows, p)
    return out.reshape(Bb, Tt * BLOCK)


# speedup vs baseline: 1.2122x; 1.2122x over previous
import jax
import jax.numpy as jnp
from jax import lax
from jax.experimental import pallas as pl
from jax.experimental.pallas import tpu as pltpu
from jax.experimental.pallas import tpu_sc as plsc

SAMPLE_RATE = 16000
BLOCK = 64
MAX_DELAY = 480
N_VOICES = 16 * 512
NUM_WORKERS = 32
VPW = N_VOICES // NUM_WORKERS
GROUPS = VPW // 16
PAD = BLOCK
UNROLL = 8


def _sc_body(noise_hbm, params_hbm, out_hbm, noise_v, burst_v, out_v,
             params_v, r_v):
    wid = lax.axis_index("s") * 2 + lax.axis_index("c")
    pltpu.sync_copy(noise_hbm.at[wid], noise_v)
    pltpu.sync_copy(params_hbm.at[wid], params_v)

    zf = jnp.zeros((16,), jnp.float32)

    def zero_pad(k, off):
        for j in range(UNROLL):
            burst_v[pl.ds(off + j * 16, 16)] = zf
        return off + UNROLL * 16

    lax.fori_loop(0, (PAD * VPW) // (UNROLL * 16), zero_pad, 0)

    def group(g, _):
        vi = g * 16 + lax.iota(jnp.int32, 16)
        pidx = vi * 4
        pa = plsc.load_gather(params_v, [pidx])
        en = plsc.load_gather(params_v, [pidx + 1])
        pt = plsc.load_gather(params_v, [pidx + 2])
        pb = plsc.load_gather(params_v, [pidx + 3])
        tau = jnp.maximum((0.0005 + pa * 0.0495) * SAMPLE_RATE, 1.0)
        rho = jnp.exp(-1.0 / tau)
        rho2 = rho * rho
        rho4 = rho2 * rho2
        rho8 = rho4 * rho4
        tilt = pt * 2.0 - 1.0
        bandwidth = 0.05 + pb * 0.95
        dly = jnp.clip((BLOCK * (0.5 + 0.5 * bandwidth)).astype(jnp.int32),
                       1, MAX_DELAY)
        cb = vi * BLOCK
        col = g * 16 + lax.iota(jnp.int32, 16)

        def step_a(k, carry):
            env, nidx, off = carry
            e1 = env * rho
            e2 = env * rho2
            e3 = e1 * rho2
            es = (env, e1, e2, e3,
                  env * rho4, e1 * rho4, e2 * rho4, e3 * rho4)
            for j in range(UNROLL):
                nz = plsc.load_gather(noise_v, [nidx + j])
                burst_v[pl.ds(off + j * VPW, 16)] = nz * es[j]
            return (env * rho8, nidx + UNROLL, off + UNROLL * VPW)

        lax.fori_loop(0, BLOCK // UNROLL, step_a,
                      (en, cb, PAD * VPW + g * 16))

        def step_b(k, carry):
            acc0, acc1, gidx, sidx, off = carry
            a0, a1 = acc0, acc1
            for j in range(UNROLL):
                b = burst_v[pl.ds(off + j * VPW, 16)]
                prev = plsc.load_gather(burst_v, [gidx + j * VPW])
                y = b + tilt * prev
                plsc.store_scatter(out_v, [sidx + j], y)
                if j % 2 == 0:
                    a0 = a0 + y * y
                else:
                    a1 = a1 + y * y
            return (a0, a1, gidx + UNROLL * VPW, sidx + UNROLL,
                    off + UNROLL * VPW)

        acc0, acc1, _, _, _ = lax.fori_loop(
            0, BLOCK // UNROLL, step_b,
            (zf, zf, (PAD - dly) * VPW + col, cb, PAD * VPW + g * 16))

        m = (acc0 + acc1) * (1.0 / BLOCK) + 1e-5
        bits = plsc.bitcast(m, jnp.int32)
        r = plsc.bitcast(0x5F3759DF - (bits >> 1), jnp.float32)
        for _ in range(4):
            r = r * (1.5 - 0.5 * m * r * r)
        r_v[pl.ds(g * 16, 16)] = r
        return 0

    lax.fori_loop(0, GROUPS, group, 0)

    def scale(g, off):
        rvec = r_v[pl.ds(g * 16, 16)]
        for j in range(16):
            rv = rvec[j]
            base = off + j * BLOCK
            for q in range(BLOCK // 16):
                sl = pl.ds(base + q * 16, 16)
                out_v[sl] = out_v[sl] * rv
        return off + 16 * BLOCK

    lax.fori_loop(0, GROUPS, scale, 0)
    pltpu.sync_copy(out_v, out_hbm.at[wid])


_sc_call = pl.kernel(
    _sc_body,
    out_type=jax.ShapeDtypeStruct((NUM_WORKERS, VPW * BLOCK), jnp.float32),
    mesh=plsc.VectorSubcoreMesh(core_axis_name="c", subcore_axis_name="s"),
    compiler_params=pltpu.CompilerParams(needs_layout_passes=False),
    scratch_types=[
        pltpu.VMEM((VPW * BLOCK,), jnp.float32),
        pltpu.VMEM(((PAD + BLOCK) * VPW,), jnp.float32),
        pltpu.VMEM((VPW * BLOCK,), jnp.float32),
        pltpu.VMEM((VPW * 4,), jnp.float32),
        pltpu.VMEM((VPW,), jnp.float32),
    ],
)


def kernel(transient_params, noise):
    Bb, Tt, _ = transient_params.shape
    p = transient_params.reshape(NUM_WORKERS, VPW * 4)
    noise_rows = noise.reshape(NUM_WORKERS, VPW * BLOCK)
    out = _sc_call(noise_rows, p)
    return out.reshape(Bb, Tt * BLOCK)


# speedup vs baseline: 1.5444x; 1.2740x over previous
import jax
import jax.numpy as jnp
from jax import lax
from jax.experimental import pallas as pl
from jax.experimental.pallas import tpu as pltpu
from jax.experimental.pallas import tpu_sc as plsc

SAMPLE_RATE = 16000
BLOCK = 64
MAX_DELAY = 480
N_VOICES = 16 * 512
NUM_WORKERS = 32
VPW = N_VOICES // NUM_WORKERS
GROUPS = VPW // 16
PAD = BLOCK
UNROLL = 8


def _sc_body(noise_hbm, params_hbm, out_hbm, noise_v, burst_v, out_v,
             params_v):
    wid = lax.axis_index("s") * 2 + lax.axis_index("c")
    pltpu.sync_copy(noise_hbm.at[wid], noise_v)
    pltpu.sync_copy(params_hbm.at[wid], params_v)

    zf = jnp.zeros((16,), jnp.float32)

    def zero_pad(k, off):
        for j in range(UNROLL):
            burst_v[pl.ds(off + j * 16, 16)] = zf
        return off + UNROLL * 16

    lax.fori_loop(0, (PAD * VPW) // (UNROLL * 16), zero_pad, 0)

    def group(g, _):
        gb = g * 16
        pa = params_v[pl.ds(gb, 16)]
        en = params_v[pl.ds(VPW + gb, 16)]
        pt = params_v[pl.ds(2 * VPW + gb, 16)]
        pb = params_v[pl.ds(3 * VPW + gb, 16)]
        tau = jnp.maximum((0.0005 + pa * 0.0495) * SAMPLE_RATE, 1.0)
        rho = jnp.exp(-1.0 / tau)
        rho2 = rho * rho
        rho4 = rho2 * rho2
        rho8 = rho4 * rho4
        tilt = pt * 2.0 - 1.0
        bandwidth = 0.05 + pb * 0.95
        dly = jnp.clip((BLOCK * (0.5 + 0.5 * bandwidth)).astype(jnp.int32),
                       1, MAX_DELAY)
        col = gb + lax.iota(jnp.int32, 16)

        def step_a(k, carry):
            env, noff, boff = carry
            e1 = env * rho
            e2 = env * rho2
            e3 = e1 * rho2
            es = (env, e1, e2, e3,
                  env * rho4, e1 * rho4, e2 * rho4, e3 * rho4)
            for j in range(UNROLL):
                nz = noise_v[pl.ds(noff + j * VPW, 16)]
                burst_v[pl.ds(boff + j * VPW, 16)] = nz * es[j]
            return (env * rho8, noff + UNROLL * VPW, boff + UNROLL * VPW)

        lax.fori_loop(0, BLOCK // UNROLL, step_a, (en, gb, PAD * VPW + gb))

        def step_b(k, carry):
            acc0, acc1, gidx, boff = carry
            a0, a1 = acc0, acc1
            for j in range(UNROLL):
                b = burst_v[pl.ds(boff + j * VPW, 16)]
                prev = plsc.load_gather(burst_v, [gidx + j * VPW])
                y = b + tilt * prev
                out_v[pl.ds(boff - PAD * VPW + j * VPW, 16)] = y
                if j % 2 == 0:
                    a0 = a0 + y * y
                else:
                    a1 = a1 + y * y
            return (a0, a1, gidx + UNROLL * VPW, boff + UNROLL * VPW)

        acc0, acc1, _, _ = lax.fori_loop(
            0, BLOCK // UNROLL, step_b,
            (zf, zf, (PAD - dly) * VPW + col, PAD * VPW + gb))

        m = (acc0 + acc1) * (1.0 / BLOCK) + 1e-5
        bits = plsc.bitcast(m, jnp.int32)
        r = plsc.bitcast(0x5F3759DF - (bits >> 1), jnp.float32)
        for _ in range(4):
            r = r * (1.5 - 0.5 * m * r * r)

        def scale(k, off):
            for j in range(UNROLL):
                sl = pl.ds(off + j * VPW, 16)
                out_v[sl] = out_v[sl] * r
            return off + UNROLL * VPW

        lax.fori_loop(0, BLOCK // UNROLL, scale, gb)
        return 0

    lax.fori_loop(0, GROUPS, group, 0)
    pltpu.sync_copy(out_v, out_hbm.at[wid])


_sc_call = pl.kernel(
    _sc_body,
    out_type=jax.ShapeDtypeStruct((NUM_WORKERS, BLOCK * VPW), jnp.float32),
    mesh=plsc.VectorSubcoreMesh(core_axis_name="c", subcore_axis_name="s"),
    compiler_params=pltpu.CompilerParams(needs_layout_passes=False),
    scratch_types=[
        pltpu.VMEM((BLOCK * VPW,), jnp.float32),
        pltpu.VMEM(((PAD + BLOCK) * VPW,), jnp.float32),
        pltpu.VMEM((BLOCK * VPW,), jnp.float32),
        pltpu.VMEM((4 * VPW,), jnp.float32),
    ],
)


def kernel(transient_params, noise):
    Bb, Tt, _ = transient_params.shape
    p = (transient_params.reshape(N_VOICES, 4).T
         .reshape(4, NUM_WORKERS, VPW).transpose(1, 0, 2)
         .reshape(NUM_WORKERS, 4 * VPW))
    noise_t = (noise.reshape(N_VOICES, BLOCK).T
               .reshape(BLOCK, NUM_WORKERS, VPW).transpose(1, 0, 2)
               .reshape(NUM_WORKERS, BLOCK * VPW))
    out3 = _sc_call(noise_t, p)
    out = (out3.reshape(NUM_WORKERS, BLOCK, VPW)
           .transpose(1, 0, 2).reshape(BLOCK, N_VOICES).T)
    return out.reshape(Bb, Tt * BLOCK)


# speedup vs baseline: 1.8937x; 1.2262x over previous
import jax
import jax.numpy as jnp
from jax import lax
from jax.experimental import pallas as pl
from jax.experimental.pallas import tpu as pltpu
from jax.experimental.pallas import tpu_sc as plsc

SAMPLE_RATE = 16000
BLOCK = 64
MAX_DELAY = 480
N_VOICES = 16 * 512
NUM_WORKERS = 32
VPW = N_VOICES // NUM_WORKERS
GROUPS = VPW // 16
PAD = BLOCK
UNROLL = 8


def _sc_body(noise_hbm, params_hbm, out_hbm, noise_v, burst_v, out_v,
             params_v):
    wid = lax.axis_index("s") * 2 + lax.axis_index("c")
    pltpu.sync_copy(noise_hbm.at[wid], noise_v)
    pltpu.sync_copy(params_hbm.at[wid], params_v)

    zf = jnp.zeros((16,), jnp.float32)

    def zero_pad(k, off):
        for j in range(UNROLL):
            burst_v[pl.ds(off + j * 16, 16)] = zf
        return off + UNROLL * 16

    lax.fori_loop(0, (PAD * VPW) // (UNROLL * 16), zero_pad, 0)

    def group(g, _):
        gb = g * 16
        pa = params_v[pl.ds(gb, 16)]
        en = params_v[pl.ds(VPW + gb, 16)]
        pt = params_v[pl.ds(2 * VPW + gb, 16)]
        pb = params_v[pl.ds(3 * VPW + gb, 16)]
        tau = jnp.maximum((0.0005 + pa * 0.0495) * SAMPLE_RATE, 1.0)
        rho = jnp.exp(-1.0 / tau)
        rho2 = rho * rho
        rho4 = rho2 * rho2
        rho8 = rho4 * rho4
        tilt = pt * 2.0 - 1.0
        bandwidth = 0.05 + pb * 0.95
        dly = jnp.clip((BLOCK * (0.5 + 0.5 * bandwidth)).astype(jnp.int32),
                       1, MAX_DELAY)
        col = gb + lax.iota(jnp.int32, 16)

        def step_a(k, carry):
            env, noff, boff = carry
            e1 = env * rho
            e2 = env * rho2
            e3 = e1 * rho2
            es = (env, e1, e2, e3,
                  env * rho4, e1 * rho4, e2 * rho4, e3 * rho4)
            nzs = [noise_v[pl.ds(noff + j * VPW, 16)] for j in range(UNROLL)]
            ys = [nzs[j] * es[j] for j in range(UNROLL)]
            for j in range(UNROLL):
                burst_v[pl.ds(boff + j * VPW, 16)] = ys[j]
            return (env * rho8, noff + UNROLL * VPW, boff + UNROLL * VPW)

        lax.fori_loop(0, BLOCK // UNROLL, step_a, (en, gb, PAD * VPW + gb))

        def step_b(k, carry):
            acc0, acc1, gidx, boff = carry
            bs = [burst_v[pl.ds(boff + j * VPW, 16)] for j in range(UNROLL)]
            prevs = [plsc.load_gather(burst_v, [gidx + j * VPW])
                     for j in range(UNROLL)]
            ys = [bs[j] + tilt * prevs[j] for j in range(UNROLL)]
            for j in range(UNROLL):
                out_v[pl.ds(boff - PAD * VPW + j * VPW, 16)] = ys[j]
            sq = [y * y for y in ys]
            a0 = acc0 + ((sq[0] + sq[1]) + (sq[2] + sq[3]))
            a1 = acc1 + ((sq[4] + sq[5]) + (sq[6] + sq[7]))
            return (a0, a1, gidx + UNROLL * VPW, boff + UNROLL * VPW)

        acc0, acc1, _, _ = lax.fori_loop(
            0, BLOCK // UNROLL, step_b,
            (zf, zf, (PAD - dly) * VPW + col, PAD * VPW + gb))

        m = (acc0 + acc1) * (1.0 / BLOCK) + 1e-5
        bits = plsc.bitcast(m, jnp.int32)
        r = plsc.bitcast(0x5F3759DF - (bits >> 1), jnp.float32)
        for _ in range(4):
            r = r * (1.5 - 0.5 * m * r * r)

        def scale(k, off):
            vs = [out_v[pl.ds(off + j * VPW, 16)] for j in range(UNROLL)]
            ws = [v * r for v in vs]
            for j in range(UNROLL):
                out_v[pl.ds(off + j * VPW, 16)] = ws[j]
            return off + UNROLL * VPW

        lax.fori_loop(0, BLOCK // UNROLL, scale, gb)
        return 0

    lax.fori_loop(0, GROUPS, group, 0)
    pltpu.sync_copy(out_v, out_hbm.at[wid])


_sc_call = pl.kernel(
    _sc_body,
    out_type=jax.ShapeDtypeStruct((NUM_WORKERS, BLOCK * VPW), jnp.float32),
    mesh=plsc.VectorSubcoreMesh(core_axis_name="c", subcore_axis_name="s"),
    compiler_params=pltpu.CompilerParams(needs_layout_passes=False),
    scratch_types=[
        pltpu.VMEM((BLOCK * VPW,), jnp.float32),
        pltpu.VMEM(((PAD + BLOCK) * VPW,), jnp.float32),
        pltpu.VMEM((BLOCK * VPW,), jnp.float32),
        pltpu.VMEM((4 * VPW,), jnp.float32),
    ],
)


def kernel(transient_params, noise):
    Bb, Tt, _ = transient_params.shape
    p = (transient_params.reshape(N_VOICES, 4).T
         .reshape(4, NUM_WORKERS, VPW).transpose(1, 0, 2)
         .reshape(NUM_WORKERS, 4 * VPW))
    noise_t = (noise.reshape(N_VOICES, BLOCK).T
               .reshape(BLOCK, NUM_WORKERS, VPW).transpose(1, 0, 2)
               .reshape(NUM_WORKERS, BLOCK * VPW))
    out3 = _sc_call(noise_t, p)
    out = (out3.reshape(NUM_WORKERS, BLOCK, VPW)
           .transpose(1, 0, 2).reshape(BLOCK, N_VOICES).T)
    return out.reshape(Bb, Tt * BLOCK)


# speedup vs baseline: 2.3043x; 1.2168x over previous
import jax
import jax.numpy as jnp
from jax import lax
from jax.experimental import pallas as pl
from jax.experimental.pallas import tpu as pltpu
from jax.experimental.pallas import tpu_sc as plsc

SAMPLE_RATE = 16000
BLOCK = 64
MAX_DELAY = 480
N_VOICES = 16 * 512
NUM_WORKERS = 32
VPW = N_VOICES // NUM_WORKERS
GROUPS = VPW // 16
PAD = BLOCK
NPITCH = VPW + 1
YPITCH = BLOCK + 1
UNROLL = 8


def _sc_body(noise_hbm, params_hbm, out_hbm, noise_v, nz_t, burst_v, y65,
             out_v, params_v, r_v):
    wid = lax.axis_index("s") * 2 + lax.axis_index("c")
    row = wid // 2
    colw = (wid % 2)
    pltpu.sync_copy(noise_hbm.at[row, pl.ds(colw * (VPW * BLOCK),
                                            VPW * BLOCK)], noise_v)
    pltpu.sync_copy(params_hbm.at[row, pl.ds(colw * (VPW * 4), VPW * 4)],
                    params_v)

    iota = lax.iota(jnp.int32, 16)
    iota_np = iota * NPITCH
    zf = jnp.zeros((16,), jnp.float32)

    def zero_pad(k, off):
        for j in range(UNROLL):
            burst_v[pl.ds(off + j * 16, 16)] = zf
        return off + UNROLL * 16

    lax.fori_loop(0, (PAD * VPW) // (UNROLL * 16), zero_pad, 0)

    def repack_in(k, carry):
        voff, base = carry
        ls = [noise_v[pl.ds(voff + j * BLOCK + q * 16, 16)]
              for j in range(2) for q in range(4)]
        i = 0
        for j in range(2):
            for q in range(4):
                plsc.store_scatter(
                    nz_t, [iota_np + (base + j + q * 16 * NPITCH)], ls[i])
                i += 1
        return (voff + 2 * BLOCK, base + 2)

    lax.fori_loop(0, VPW // 2, repack_in, (0, 0))

    def group(g, _):
        gb = g * 16
        vi = gb + iota
        pidx = vi * 4
        pa = plsc.load_gather(params_v, [pidx])
        en = plsc.load_gather(params_v, [pidx + 1])
        pt = plsc.load_gather(params_v, [pidx + 2])
        pb = plsc.load_gather(params_v, [pidx + 3])
        tau = jnp.maximum((0.0005 + pa * 0.0495) * SAMPLE_RATE, 1.0)
        rho = jnp.exp(-1.0 / tau)
        rho2 = rho * rho
        rho4 = rho2 * rho2
        rho8 = rho4 * rho4
        tilt = pt * 2.0 - 1.0
        bandwidth = 0.05 + pb * 0.95
        dly = jnp.clip((BLOCK * (0.5 + 0.5 * bandwidth)).astype(jnp.int32),
                       1, MAX_DELAY)

        def step_a(k, carry):
            env, noff, boff = carry
            e1 = env * rho
            e2 = env * rho2
            e3 = e1 * rho2
            es = (env, e1, e2, e3,
                  env * rho4, e1 * rho4, e2 * rho4, e3 * rho4)
            nzs = [nz_t[pl.ds(noff + j * NPITCH, 16)] for j in range(UNROLL)]
            ys = [nzs[j] * es[j] for j in range(UNROLL)]
            for j in range(UNROLL):
                burst_v[pl.ds(boff + j * VPW, 16)] = ys[j]
            return (env * rho8, noff + UNROLL * NPITCH, boff + UNROLL * VPW)

        lax.fori_loop(0, BLOCK // UNROLL, step_a, (en, gb, PAD * VPW + gb))

        def step_b(k, carry):
            acc0, acc1, gidx, boff, yv = carry
            bs = [burst_v[pl.ds(boff + j * VPW, 16)] for j in range(UNROLL)]
            prevs = [plsc.load_gather(burst_v, [gidx + j * VPW])
                     for j in range(UNROLL)]
            ys = [bs[j] + tilt * prevs[j] for j in range(UNROLL)]
            for j in range(UNROLL):
                plsc.store_scatter(y65, [yv + j], ys[j])
            sq = [y * y for y in ys]
            a0 = acc0 + ((sq[0] + sq[1]) + (sq[2] + sq[3]))
            a1 = acc1 + ((sq[4] + sq[5]) + (sq[6] + sq[7]))
            return (a0, a1, gidx + UNROLL * VPW, boff + UNROLL * VPW,
                    yv + UNROLL)

        acc0, acc1, _, _, _ = lax.fori_loop(
            0, BLOCK // UNROLL, step_b,
            (zf, zf, (PAD - dly) * VPW + vi, PAD * VPW + gb, vi * YPITCH))

        m = (acc0 + acc1) * (1.0 / BLOCK) + 1e-5
        bits = plsc.bitcast(m, jnp.int32)
        r = plsc.bitcast(0x5F3759DF - (bits >> 1), jnp.float32)
        for _ in range(4):
            r = r * (1.5 - 0.5 * m * r * r)
        r_v[pl.ds(gb, 16)] = r
        return 0

    lax.fori_loop(0, GROUPS, group, 0)

    def repack_out(g, _):
        gb = g * 16
        rvec = r_v[pl.ds(gb, 16)]
        for j in range(0, 16, 2):
            rv0 = rvec[j]
            rv1 = rvec[j + 1]
            b0 = (gb + j) * YPITCH
            b1 = (gb + j + 1) * YPITCH
            o0 = (gb + j) * BLOCK
            o1 = (gb + j + 1) * BLOCK
            vs = ([plsc.load_gather(y65, [iota + (b0 + q * 16)])
                   for q in range(4)] +
                  [plsc.load_gather(y65, [iota + (b1 + q * 16)])
                   for q in range(4)])
            ws = [v * rv0 for v in vs[:4]] + [v * rv1 for v in vs[4:]]
            for q in range(4):
                out_v[pl.ds(o0 + q * 16, 16)] = ws[q]
            for q in range(4):
                out_v[pl.ds(o1 + q * 16, 16)] = ws[4 + q]
        return 0

    lax.fori_loop(0, GROUPS, repack_out, 0)
    pltpu.sync_copy(out_v, out_hbm.at[row, pl.ds(colw * (VPW * BLOCK),
                                                 VPW * BLOCK)])


_sc_call = pl.kernel(
    _sc_body,
    out_type=jax.ShapeDtypeStruct((N_VOICES // 512, 512 * BLOCK),
                                  jnp.float32),
    mesh=plsc.VectorSubcoreMesh(core_axis_name="c", subcore_axis_name="s"),
    compiler_params=pltpu.CompilerParams(needs_layout_passes=False),
    scratch_types=[
        pltpu.VMEM((VPW * BLOCK,), jnp.float32),
        pltpu.VMEM((BLOCK * NPITCH,), jnp.float32),
        pltpu.VMEM(((PAD + BLOCK) * VPW,), jnp.float32),
        pltpu.VMEM((VPW * YPITCH,), jnp.float32),
        pltpu.VMEM((VPW * BLOCK,), jnp.float32),
        pltpu.VMEM((4 * VPW,), jnp.float32),
        pltpu.VMEM((VPW,), jnp.float32),
    ],
)


def kernel(transient_params, noise):
    Bb, Tt, _ = transient_params.shape
    p = transient_params.reshape(Bb, Tt * 4)
    noise_rows = noise.reshape(Bb, Tt * BLOCK)
    return _sc_call(noise_rows, p)


# speedup vs baseline: 2.3941x; 1.0390x over previous
import jax
import jax.numpy as jnp
from jax import lax
from jax.experimental import pallas as pl
from jax.experimental.pallas import tpu as pltpu
from jax.experimental.pallas import tpu_sc as plsc

SAMPLE_RATE = 16000
BLOCK = 64
MAX_DELAY = 480
N_VOICES = 16 * 512
NUM_WORKERS = 32
VPW = N_VOICES // NUM_WORKERS
GROUPS = VPW // 16
PAD = BLOCK
NPITCH = VPW + 1
YPITCH = BLOCK + 1
UNROLL = 8


def _sc_body(noise_hbm, params_hbm, out_hbm, noise_v, nz_t, burst_v, y65,
             out_v, params_v, r_v, sem_in, sem_out):
    wid = lax.axis_index("s") * 2 + lax.axis_index("c")
    row = wid // 2
    colw = (wid % 2)
    cp_noise = pltpu.async_copy(
        noise_hbm.at[row, pl.ds(colw * (VPW * BLOCK), VPW * BLOCK)],
        noise_v, sem_in)
    cp_params = pltpu.async_copy(
        params_hbm.at[row, pl.ds(colw * (VPW * 4), VPW * 4)],
        params_v, sem_in)

    iota = lax.iota(jnp.int32, 16)
    iota_np = iota * NPITCH
    zf = jnp.zeros((16,), jnp.float32)

    def zero_pad(k, off):
        for j in range(UNROLL):
            burst_v[pl.ds(off + j * 16, 16)] = zf
        return off + UNROLL * 16

    lax.fori_loop(0, (PAD * VPW) // (UNROLL * 16), zero_pad, 0)
    cp_noise.wait()
    cp_params.wait()

    def repack_in(k, carry):
        voff, base = carry
        ls = [noise_v[pl.ds(voff + j * BLOCK + q * 16, 16)]
              for j in range(2) for q in range(4)]
        i = 0
        for j in range(2):
            for q in range(4):
                plsc.store_scatter(
                    nz_t, [iota_np + (base + j + q * 16 * NPITCH)], ls[i])
                i += 1
        return (voff + 2 * BLOCK, base + 2)

    lax.fori_loop(0, VPW // 2, repack_in, (0, 0))

    def group(g, _):
        gb = g * 16
        vi = gb + iota
        pidx = vi * 4
        pa = plsc.load_gather(params_v, [pidx])
        en = plsc.load_gather(params_v, [pidx + 1])
        pt = plsc.load_gather(params_v, [pidx + 2])
        pb = plsc.load_gather(params_v, [pidx + 3])
        tau = jnp.maximum((0.0005 + pa * 0.0495) * SAMPLE_RATE, 1.0)
        rho = jnp.exp(-1.0 / tau)
        rho2 = rho * rho
        rho4 = rho2 * rho2
        rho8 = rho4 * rho4
        tilt = pt * 2.0 - 1.0
        bandwidth = 0.05 + pb * 0.95
        dly = jnp.clip((BLOCK * (0.5 + 0.5 * bandwidth)).astype(jnp.int32),
                       1, MAX_DELAY)

        def step_a(k, carry):
            env, noff, boff = carry
            e1 = env * rho
            e2 = env * rho2
            e3 = e1 * rho2
            es = (env, e1, e2, e3,
                  env * rho4, e1 * rho4, e2 * rho4, e3 * rho4)
            nzs = [nz_t[pl.ds(noff + j * NPITCH, 16)] for j in range(UNROLL)]
            ys = [nzs[j] * es[j] for j in range(UNROLL)]
            for j in range(UNROLL):
                burst_v[pl.ds(boff + j * VPW, 16)] = ys[j]
            return (env * rho8, noff + UNROLL * NPITCH, boff + UNROLL * VPW)

        lax.fori_loop(0, BLOCK // UNROLL, step_a, (en, gb, PAD * VPW + gb))

        def step_b(k, carry):
            acc0, acc1, gidx, boff, yv = carry
            bs = [burst_v[pl.ds(boff + j * VPW, 16)] for j in range(UNROLL)]
            prevs = [plsc.load_gather(burst_v, [gidx + j * VPW])
                     for j in range(UNROLL)]
            ys = [bs[j] + tilt * prevs[j] for j in range(UNROLL)]
            for j in range(UNROLL):
                plsc.store_scatter(y65, [yv + j], ys[j])
            sq = [y * y for y in ys]
            a0 = acc0 + ((sq[0] + sq[1]) + (sq[2] + sq[3]))
            a1 = acc1 + ((sq[4] + sq[5]) + (sq[6] + sq[7]))
            return (a0, a1, gidx + UNROLL * VPW, boff + UNROLL * VPW,
                    yv + UNROLL)

        acc0, acc1, _, _, _ = lax.fori_loop(
            0, BLOCK // UNROLL, step_b,
            (zf, zf, (PAD - dly) * VPW + vi, PAD * VPW + gb, vi * YPITCH))

        m = (acc0 + acc1) * (1.0 / BLOCK) + 1e-5
        bits = plsc.bitcast(m, jnp.int32)
        r = plsc.bitcast(0x5F3759DF - (bits >> 1), jnp.float32)
        for _ in range(4):
            r = r * (1.5 - 0.5 * m * r * r)
        r_v[pl.ds(gb, 16)] = r
        return 0

    lax.fori_loop(0, GROUPS, group, 0)

    def repack_out(g, _):
        gb = g * 16
        rvec = r_v[pl.ds(gb, 16)]
        for j in range(0, 16, 2):
            rv0 = rvec[j]
            rv1 = rvec[j + 1]
            b0 = (gb + j) * YPITCH
            b1 = (gb + j + 1) * YPITCH
            o0 = (gb + j) * BLOCK
            o1 = (gb + j + 1) * BLOCK
            vs = ([plsc.load_gather(y65, [iota + (b0 + q * 16)])
                   for q in range(4)] +
                  [plsc.load_gather(y65, [iota + (b1 + q * 16)])
                   for q in range(4)])
            ws = [v * rv0 for v in vs[:4]] + [v * rv1 for v in vs[4:]]
            for q in range(4):
                out_v[pl.ds(o0 + q * 16, 16)] = ws[q]
            for q in range(4):
                out_v[pl.ds(o1 + q * 16, 16)] = ws[4 + q]
        return 0

    chunk = (VPW * BLOCK) // 4
    handles = []
    for c in range(4):
        lax.fori_loop(c * (GROUPS // 4), (c + 1) * (GROUPS // 4),
                      repack_out, 0)
        handles.append(pltpu.async_copy(
            out_v.at[pl.ds(c * chunk, chunk)],
            out_hbm.at[row, pl.ds(colw * (VPW * BLOCK) + c * chunk, chunk)],
            sem_out))
    for h in handles:
        h.wait()


_sc_call = pl.kernel(
    _sc_body,
    out_type=jax.ShapeDtypeStruct((N_VOICES // 512, 512 * BLOCK),
                                  jnp.float32),
    mesh=plsc.VectorSubcoreMesh(core_axis_name="c", subcore_axis_name="s"),
    compiler_params=pltpu.CompilerParams(needs_layout_passes=False),
    scratch_types=[
        pltpu.VMEM((VPW * BLOCK,), jnp.float32),
        pltpu.VMEM((BLOCK * NPITCH,), jnp.float32),
        pltpu.VMEM(((PAD + BLOCK) * VPW,), jnp.float32),
        pltpu.VMEM((VPW * YPITCH,), jnp.float32),
        pltpu.VMEM((VPW * BLOCK,), jnp.float32),
        pltpu.VMEM((4 * VPW,), jnp.float32),
        pltpu.VMEM((VPW,), jnp.float32),
        pltpu.SemaphoreType.DMA,
        pltpu.SemaphoreType.DMA,
    ],
)


def kernel(transient_params, noise):
    Bb, Tt, _ = transient_params.shape
    p = transient_params.reshape(Bb, Tt * 4)
    noise_rows = noise.reshape(Bb, Tt * BLOCK)
    return _sc_call(noise_rows, p)


# speedup vs baseline: 2.4515x; 1.0240x over previous
import jax
import jax.numpy as jnp
from jax import lax
from jax.experimental import pallas as pl
from jax.experimental.pallas import tpu as pltpu
from jax.experimental.pallas import tpu_sc as plsc

SAMPLE_RATE = 16000
BLOCK = 64
MAX_DELAY = 480
N_VOICES = 16 * 512
NUM_WORKERS = 32
VPW = N_VOICES // NUM_WORKERS
GROUPS = VPW // 16
PAD = BLOCK
NPITCH = VPW + 1
YPITCH = BLOCK + 1
UNROLL = 8


def _sc_body(noise_hbm, params_hbm, out_hbm, noise_v, nz_t, y65,
             out_v, params_v, r_v, sem_in, sem_out):
    wid = lax.axis_index("s") * 2 + lax.axis_index("c")
    row = wid // 2
    colw = (wid % 2)
    cp_noise = pltpu.async_copy(
        noise_hbm.at[row, pl.ds(colw * (VPW * BLOCK), VPW * BLOCK)],
        noise_v, sem_in)
    cp_params = pltpu.async_copy(
        params_hbm.at[row, pl.ds(colw * (VPW * 4), VPW * 4)],
        params_v, sem_in)

    iota = lax.iota(jnp.int32, 16)
    iota_np = iota * NPITCH
    zf = jnp.zeros((16,), jnp.float32)

    def zero_pad(k, off):
        for j in range(UNROLL):
            nz_t[pl.ds(off + j * 16, 16)] = zf
        return off + UNROLL * 16

    lax.fori_loop(0, (PAD * NPITCH + UNROLL * 16 - 1) // (UNROLL * 16),
                  zero_pad, 0)
    cp_noise.wait()
    cp_params.wait()

    def repack_in(k, carry):
        voff, base = carry
        ls = [noise_v[pl.ds(voff + j * BLOCK + q * 16, 16)]
              for j in range(2) for q in range(4)]
        i = 0
        for j in range(2):
            for q in range(4):
                plsc.store_scatter(
                    nz_t,
                    [iota_np + (base + j + (PAD + q * 16) * NPITCH)], ls[i])
                i += 1
        return (voff + 2 * BLOCK, base + 2)

    lax.fori_loop(0, VPW // 2, repack_in, (0, 0))

    def group(g, _):
        gb = g * 16
        vi = gb + iota
        pidx = vi * 4
        pa = plsc.load_gather(params_v, [pidx])
        en = plsc.load_gather(params_v, [pidx + 1])
        pt = plsc.load_gather(params_v, [pidx + 2])
        pb = plsc.load_gather(params_v, [pidx + 3])
        tau = jnp.maximum((0.0005 + pa * 0.0495) * SAMPLE_RATE, 1.0)
        rho = jnp.exp(-1.0 / tau)
        rho2 = rho * rho
        rho4 = rho2 * rho2
        rho8 = rho4 * rho4
        tilt = pt * 2.0 - 1.0
        bandwidth = 0.05 + pb * 0.95
        dly = jnp.clip((BLOCK * (0.5 + 0.5 * bandwidth)).astype(jnp.int32),
                       1, MAX_DELAY)
        tilt2 = tilt * jnp.exp(dly.astype(jnp.float32) / tau)

        def step_b(k, carry):
            acc0, acc1, env, gidx, noff, yv = carry
            e1 = env * rho
            e2 = env * rho2
            e3 = e1 * rho2
            es = (env, e1, e2, e3,
                  env * rho4, e1 * rho4, e2 * rho4, e3 * rho4)
            bs = [nz_t[pl.ds(noff + j * NPITCH, 16)] for j in range(UNROLL)]
            gs = [plsc.load_gather(nz_t, [gidx + j * NPITCH])
                  for j in range(UNROLL)]
            ys = [es[j] * (bs[j] + tilt2 * gs[j]) for j in range(UNROLL)]
            for j in range(UNROLL):
                plsc.store_scatter(y65, [yv + j], ys[j])
            sq = [y * y for y in ys]
            a0 = acc0 + ((sq[0] + sq[1]) + (sq[2] + sq[3]))
            a1 = acc1 + ((sq[4] + sq[5]) + (sq[6] + sq[7]))
            return (a0, a1, env * rho8, gidx + UNROLL * NPITCH,
                    noff + UNROLL * NPITCH, yv + UNROLL)

        acc0, acc1, _, _, _, _ = lax.fori_loop(
            0, BLOCK // UNROLL, step_b,
            (zf, zf, en, (PAD - dly) * NPITCH + vi, PAD * NPITCH + gb,
             vi * YPITCH))

        m = (acc0 + acc1) * (1.0 / BLOCK) + 1e-5
        bits = plsc.bitcast(m, jnp.int32)
        r = plsc.bitcast(0x5F3759DF - (bits >> 1), jnp.float32)
        for _ in range(4):
            r = r * (1.5 - 0.5 * m * r * r)
        r_v[pl.ds(gb, 16)] = r
        return 0

    lax.fori_loop(0, GROUPS, group, 0)

    def repack_out(g, _):
        gb = g * 16
        rvec = r_v[pl.ds(gb, 16)]
        for j in range(0, 16, 2):
            rv0 = rvec[j]
            rv1 = rvec[j + 1]
            b0 = (gb + j) * YPITCH
            b1 = (gb + j + 1) * YPITCH
            o0 = (gb + j) * BLOCK
            o1 = (gb + j + 1) * BLOCK
            vs = ([plsc.load_gather(y65, [iota + (b0 + q * 16)])
                   for q in range(4)] +
                  [plsc.load_gather(y65, [iota + (b1 + q * 16)])
                   for q in range(4)])
            ws = [v * rv0 for v in vs[:4]] + [v * rv1 for v in vs[4:]]
            for q in range(4):
                out_v[pl.ds(o0 + q * 16, 16)] = ws[q]
            for q in range(4):
                out_v[pl.ds(o1 + q * 16, 16)] = ws[4 + q]
        return 0

    chunk = (VPW * BLOCK) // 4
    handles = []
    for c in range(4):
        lax.fori_loop(c * (GROUPS // 4), (c + 1) * (GROUPS // 4),
                      repack_out, 0)
        handles.append(pltpu.async_copy(
            out_v.at[pl.ds(c * chunk, chunk)],
            out_hbm.at[row, pl.ds(colw * (VPW * BLOCK) + c * chunk, chunk)],
            sem_out))
    for h in handles:
        h.wait()


_sc_call = pl.kernel(
    _sc_body,
    out_type=jax.ShapeDtypeStruct((N_VOICES // 512, 512 * BLOCK),
                                  jnp.float32),
    mesh=plsc.VectorSubcoreMesh(core_axis_name="c", subcore_axis_name="s"),
    compiler_params=pltpu.CompilerParams(needs_layout_passes=False),
    scratch_types=[
        pltpu.VMEM((VPW * BLOCK,), jnp.float32),
        pltpu.VMEM(((PAD + BLOCK) * NPITCH,), jnp.float32),
        pltpu.VMEM((VPW * YPITCH,), jnp.float32),
        pltpu.VMEM((VPW * BLOCK,), jnp.float32),
        pltpu.VMEM((4 * VPW,), jnp.float32),
        pltpu.VMEM((VPW,), jnp.float32),
        pltpu.SemaphoreType.DMA,
        pltpu.SemaphoreType.DMA,
    ],
)


def kernel(transient_params, noise):
    Bb, Tt, _ = transient_params.shape
    p = transient_params.reshape(Bb, Tt * 4)
    noise_rows = noise.reshape(Bb, Tt * BLOCK)
    return _sc_call(noise_rows, p)
